# baseline (speedup 1.0000x reference)
import functools

import jax
import jax.numpy as jnp
from jax import lax
from jax.experimental import pallas as pl
from jax.experimental.pallas import tpu as pltpu
from jax.experimental.pallas import tpu_sc as plsc

_NBUF = 3
_NOBUF = 3
_LANES = 128


@functools.lru_cache(maxsize=None)
def _build_gather(B0, S, V, D):
    info = plsc.get_sparse_core_info()
    nc, ns = info.num_cores, info.num_subcores
    nw = nc * ns
    assert B0 % (nw * _LANES) == 0 and D % 16 == 0
    n = S
    dh = D // 8
    sp = (S + 7) // 8 * 8
    mesh = plsc.VectorSubcoreMesh(core_axis_name="c", subcore_axis_name="s")

    @functools.partial(
        pl.kernel,
        mesh=mesh,
        out_type=jax.ShapeDtypeStruct((S, 8, B0 // _LANES, dh, _LANES),
                                      jnp.float32),
        scratch_types=[
            pltpu.VMEM((sp, _LANES), jnp.int32),
            [pltpu.VMEM((_LANES, D), jnp.float32) for _ in range(_NBUF)],
            [pltpu.VMEM((D, _LANES), jnp.float32) for _ in range(_NOBUF)],
            pltpu.SemaphoreType.DMA,
            pltpu.SemaphoreType.DMA,
        ],
        compiler_params=pltpu.CompilerParams(
            use_tc_tiling_on_sc=False, needs_layout_passes=False
        ),
    )
    def k(idx_hbm, table_hbm, out_hbm, idx_v, gbufs, obufs, sem_g, sem_o):
        wid = lax.axis_index("s") * nc + lax.axis_index("c")
        ih = wid
        i0 = wid * _LANES
        pltpu.sync_copy(idx_hbm.at[:, pl.ds(i0, _LANES)], idx_v)

        def gather(j, g):
            pltpu.async_copy(table_hbm.at[idx_v.at[j]], g, sem_g)

        def wait_gather(j, g):
            pltpu.make_async_copy(table_hbm.at[idx_v.at[j]], g, sem_g).wait()

        def put(j, o):
            for d8 in range(8):
                pltpu.async_copy(
                    o.at[pl.ds(d8 * dh, dh)], out_hbm.at[j, d8, ih], sem_o
                )

        def wait_put(j, o):
            for d8 in range(8):
                pltpu.make_async_copy(
                    o.at[pl.ds(d8 * dh, dh)], out_hbm.at[j, d8, ih], sem_o
                ).wait()

        def vtrans(g, o):
            def body(i4, carry):
                rows = [
                    lax.iota(jnp.int32, 16) + (16 * c) for c in range(D // 16)
                ]
                for u in range(4):
                    i = i4 * 4 + u
                    col = jnp.full((16,), i, jnp.int32)
                    for c in range(D // 16):
                        plsc.store_scatter(
                            o, [rows[c], col], g[i, pl.ds(c * 16, 16)]
                        )
                return carry

            lax.fori_loop(0, _LANES // 4, body, 0)

        def step(j, b, ob, do_gather=True, do_retire=True):
            wait_gather(j, gbufs[b])
            if do_gather:
                gather(j + _NBUF - 1, gbufs[(b + _NBUF - 1) % _NBUF])
            if do_retire:
                wait_put(j - _NOBUF, obufs[ob])
            vtrans(gbufs[b], obufs[ob])
            put(j, obufs[ob])

        for j in range(_NBUF - 1):
            gather(j, gbufs[j])

        for j in range(_NOBUF):
            step(j, j % _NBUF, j % _NOBUF, do_retire=False)

        grp = _NBUF * _NOBUF
        g0 = _NOBUF
        nsteady = (n - g0 - (_NBUF - 1) - (n - g0 - (_NBUF - 1)) % grp)
        for j in range(g0, g0 + (n - g0 - (_NBUF - 1)) % grp):
            step(j, j % _NBUF, j % _NOBUF)
        g1 = g0 + (n - g0 - (_NBUF - 1)) % grp

        def group(t, carry):
            for u in range(grp):
                j = g1 + t * grp + u
                step(j, (g1 + u) % _NBUF, (g1 + u) % _NOBUF)
            return carry

        lax.fori_loop(0, nsteady // grp, group, 0, unroll=False)

        for j in range(g1 + nsteady, n):
            step(j, j % _NBUF, j % _NOBUF, do_gather=(j + _NBUF - 1 < n))

        for j in range(n - _NOBUF, n):
            wait_put(j, obufs[j % _NOBUF])

    return k


def kernel(inputs, embeddings):
    B0, S = inputs.shape
    V, D = embeddings.shape
    sp = (S + 7) // 8 * 8
    idx_t = jnp.pad(jnp.transpose(inputs.astype(jnp.int32)),
                    ((0, sp - S), (0, 0)))
    out5 = _build_gather(B0, S, V, D)(idx_t, embeddings)
    return out5.transpose(2, 4, 0, 1, 3).reshape(B0, S, D)


# speedup vs baseline: 1.8158x; 1.8158x over previous
import functools

import jax
import jax.numpy as jnp
from jax import lax
from jax.experimental import pallas as pl
from jax.experimental.pallas import tpu as pltpu
from jax.experimental.pallas import tpu_sc as plsc

_NBUF = 3
_NOBUF = 3
_LANES = 128


@functools.lru_cache(maxsize=None)
def _build_gather(B0, S, V, D):
    info = plsc.get_sparse_core_info()
    nc, ns = info.num_cores, info.num_subcores
    nw = nc * ns
    assert B0 % (nw * _LANES) == 0 and D % 16 == 0
    n = S
    dh = D // 8
    sp = (S + 7) // 8 * 8
    mesh = plsc.VectorSubcoreMesh(core_axis_name="c", subcore_axis_name="s")

    @functools.partial(
        pl.kernel,
        mesh=mesh,
        out_type=jax.ShapeDtypeStruct((S, 8, B0 // _LANES, dh, _LANES),
                                      jnp.float32),
        scratch_types=[
            pltpu.VMEM((sp, _LANES), jnp.int32),
            [pltpu.VMEM((_LANES, D), jnp.float32) for _ in range(_NBUF)],
            [pltpu.VMEM((D, _LANES + 1), jnp.float32) for _ in range(_NOBUF)],
            pltpu.SemaphoreType.DMA,
            pltpu.SemaphoreType.DMA,
        ],
        compiler_params=pltpu.CompilerParams(
            use_tc_tiling_on_sc=False, needs_layout_passes=False
        ),
    )
    def k(idx_hbm, table_hbm, out_hbm, idx_v, gbufs, obufs, sem_g, sem_o):
        wid = lax.axis_index("s") * nc + lax.axis_index("c")
        ih = wid
        i0 = wid * _LANES
        pltpu.sync_copy(idx_hbm.at[:, pl.ds(i0, _LANES)], idx_v)

        def gather(j, g):
            pltpu.async_copy(table_hbm.at[idx_v.at[j]], g, sem_g)

        def wait_gather(j, g):
            pltpu.make_async_copy(table_hbm.at[idx_v.at[j]], g, sem_g).wait()

        def put(j, o):
            for d8 in range(8):
                pltpu.async_copy(
                    o.at[pl.ds(d8 * dh, dh), pl.ds(0, _LANES)],
                    out_hbm.at[j, d8, ih],
                    sem_o,
                )

        def wait_put(j, o):
            for d8 in range(8):
                pltpu.make_async_copy(
                    o.at[pl.ds(d8 * dh, dh), pl.ds(0, _LANES)],
                    out_hbm.at[j, d8, ih],
                    sem_o,
                ).wait()

        def vtrans(g, o):
            def body(i4, carry):
                rows = [
                    lax.iota(jnp.int32, 16) + (16 * c) for c in range(D // 16)
                ]
                for u in range(4):
                    i = i4 * 4 + u
                    col = jnp.full((16,), i, jnp.int32)
                    for c in range(D // 16):
                        plsc.store_scatter(
                            o, [rows[c], col], g[i, pl.ds(c * 16, 16)]
                        )
                return carry

            lax.fori_loop(0, _LANES // 4, body, 0)

        def step(j, b, ob, do_gather=True, do_retire=True):
            wait_gather(j, gbufs[b])
            if do_gather:
                gather(j + _NBUF - 1, gbufs[(b + _NBUF - 1) % _NBUF])
            if do_retire:
                wait_put(j - _NOBUF, obufs[ob])
            vtrans(gbufs[b], obufs[ob])
            put(j, obufs[ob])

        for j in range(_NBUF - 1):
            gather(j, gbufs[j])

        for j in range(_NOBUF):
            step(j, j % _NBUF, j % _NOBUF, do_retire=False)

        grp = _NBUF * _NOBUF
        g0 = _NOBUF
        nsteady = (n - g0 - (_NBUF - 1) - (n - g0 - (_NBUF - 1)) % grp)
        for j in range(g0, g0 + (n - g0 - (_NBUF - 1)) % grp):
            step(j, j % _NBUF, j % _NOBUF)
        g1 = g0 + (n - g0 - (_NBUF - 1)) % grp

        def group(t, carry):
            for u in range(grp):
                j = g1 + t * grp + u
                step(j, (g1 + u) % _NBUF, (g1 + u) % _NOBUF)
            return carry

        lax.fori_loop(0, nsteady // grp, group, 0, unroll=False)

        for j in range(g1 + nsteady, n):
            step(j, j % _NBUF, j % _NOBUF, do_gather=(j + _NBUF - 1 < n))

        for j in range(n - _NOBUF, n):
            wait_put(j, obufs[j % _NOBUF])

    return k


def kernel(inputs, embeddings):
    B0, S = inputs.shape
    V, D = embeddings.shape
    sp = (S + 7) // 8 * 8
    idx_t = jnp.pad(jnp.transpose(inputs.astype(jnp.int32)),
                    ((0, sp - S), (0, 0)))
    out5 = _build_gather(B0, S, V, D)(idx_t, embeddings)
    return out5.transpose(2, 4, 0, 1, 3).reshape(B0, S, D)
